# baseline (speedup 1.0000x reference)
import functools

import jax
import jax.numpy as jnp
from jax import lax
from jax.experimental import pallas as pl
from jax.experimental.pallas import tpu as pltpu
from jax.experimental.pallas import tpu_sc as plsc

N = 10000
D = 256
H = 256
E = 160000

NP = 10240
EP = 163840
TRASH = NP - 1
NTILES = 16
TB = EP // NTILES
B = 128
NB = TB // B
NBH = NB // 2
SLAB = NP // NTILES
BLK = 512


def _sc_mesh():
    return plsc.VectorSubcoreMesh(core_axis_name="c", subcore_axis_name="s")


def _deg_kernel(dst_hbm, ones_hbm, zslab_hbm, deg_out, dst_v, ones_v, acc,
                ss0, ss1, ss2, ss3):
    c = lax.axis_index("c")
    s = lax.axis_index("s")
    pltpu.sync_copy(dst_hbm.at[s, pl.ds(c * NBH, NBH)], dst_v)
    pltpu.sync_copy(ones_hbm, ones_v)
    pltpu.sync_copy(zslab_hbm, acc.at[pl.ds(s * SLAB, SLAB)])
    plsc.subcore_barrier()

    def body(gi, carry):
        base = gi * 4
        h0 = pltpu.async_copy(ones_v, acc.at[dst_v.at[base]], ss0, add=True)
        h1 = pltpu.async_copy(ones_v, acc.at[dst_v.at[base + 1]], ss1,
                              add=True)
        h2 = pltpu.async_copy(ones_v, acc.at[dst_v.at[base + 2]], ss2,
                              add=True)
        h3 = pltpu.async_copy(ones_v, acc.at[dst_v.at[base + 3]], ss3,
                              add=True)
        h0.wait()
        h1.wait()
        h2.wait()
        h3.wait()
        return carry

    lax.fori_loop(0, NBH // 4, body, 0)
    plsc.subcore_barrier()
    pltpu.sync_copy(acc.at[pl.ds(s * SLAB, SLAB)],
                    deg_out.at[c, pl.ds(s * SLAB, SLAB)])


def _run_deg(dst_tiles, ones128, zslab128):
    f = functools.partial(
        pl.kernel,
        out_type=jax.ShapeDtypeStruct((2, NP, 128), jnp.float32),
        mesh=_sc_mesh(),
        scratch_types=[
            pltpu.VMEM((NBH, B), jnp.int32),
            pltpu.VMEM((B, 128), jnp.float32),
            pltpu.VMEM_SHARED((NP, 128), jnp.float32),
            pltpu.SemaphoreType.DMA,
            pltpu.SemaphoreType.DMA,
            pltpu.SemaphoreType.DMA,
            pltpu.SemaphoreType.DMA,
        ],
    )(_deg_kernel)
    return f(dst_tiles, ones128, zslab128)


def _scatter_kernel(srco_hbm, src_hbm, dst_hbm, gflat_hbm, dinv_hbm,
                    zslab_hbm, s_out, u_out,
                    src_v, dst_v, r0, r1, acc, g0s, g1s, s0s, s1s):
    c = lax.axis_index("c")
    s = lax.axis_index("s")
    pltpu.sync_copy(zslab_hbm, acc.at[pl.ds(s * SLAB, SLAB)])
    plsc.subcore_barrier()

    for half in range(2):
        pltpu.sync_copy(srco_hbm.at[c, s, pl.ds(half * NBH, NBH)], src_v)
        pltpu.sync_copy(dst_hbm.at[s, pl.ds(half * NBH, NBH)], dst_v)

        pltpu.async_copy(gflat_hbm.at[src_v.at[0]], r0, g0s)
        pltpu.async_copy(gflat_hbm.at[src_v.at[1]], r1, g1s)

        def body(gi, carry):
            base = gi * 2
            pltpu.make_async_copy(gflat_hbm.at[src_v.at[base]], r0,
                                  g0s).wait()
            w0 = pltpu.async_copy(r0, acc.at[dst_v.at[base]], s0s, add=True)
            pltpu.make_async_copy(gflat_hbm.at[src_v.at[base + 1]], r1,
                                  g1s).wait()
            w1 = pltpu.async_copy(r1, acc.at[dst_v.at[base + 1]], s1s,
                                  add=True)
            w0.wait()

            @pl.when(base + 2 < NBH)
            def _():
                pltpu.async_copy(gflat_hbm.at[src_v.at[base + 2]], r0, g0s)

            w1.wait()

            @pl.when(base + 3 < NBH)
            def _():
                pltpu.async_copy(gflat_hbm.at[src_v.at[base + 3]], r1, g1s)

            return carry

        lax.fori_loop(0, NBH // 2, body, 0)

    plsc.subcore_barrier()
    pltpu.sync_copy(acc.at[pl.ds(s * SLAB, SLAB)],
                    s_out.at[c, pl.ds(s * SLAB, SLAB)])
    pltpu.sync_copy(zslab_hbm, acc.at[pl.ds(s * SLAB, SLAB)])
    plsc.subcore_barrier()

    pltpu.sync_copy(src_hbm.at[s, pl.ds(c * NBH, NBH)], src_v)
    pltpu.sync_copy(dst_hbm.at[s, pl.ds(c * NBH, NBH)], dst_v)

    pltpu.async_copy(dinv_hbm.at[dst_v.at[0]], r0, g0s)
    pltpu.async_copy(dinv_hbm.at[dst_v.at[1]], r1, g1s)

    def ubody(gi, carry):
        base = gi * 2
        pltpu.make_async_copy(dinv_hbm.at[dst_v.at[base]], r0, g0s).wait()
        w0 = pltpu.async_copy(r0, acc.at[src_v.at[base]], s0s, add=True)
        pltpu.make_async_copy(dinv_hbm.at[dst_v.at[base + 1]], r1,
                              g1s).wait()
        w1 = pltpu.async_copy(r1, acc.at[src_v.at[base + 1]], s1s,
                              add=True)
        w0.wait()

        @pl.when(base + 2 < NBH)
        def _():
            pltpu.async_copy(dinv_hbm.at[dst_v.at[base + 2]], r0, g0s)

        w1.wait()

        @pl.when(base + 3 < NBH)
        def _():
            pltpu.async_copy(dinv_hbm.at[dst_v.at[base + 3]], r1, g1s)

        return carry

    lax.fori_loop(0, NBH // 2, ubody, 0)
    plsc.subcore_barrier()
    pltpu.sync_copy(acc.at[pl.ds(s * SLAB, SLAB)],
                    u_out.at[c, pl.ds(s * SLAB, SLAB)])


def _run_scatter(src_tiles2, src_tiles, dst_tiles, gflat, dinv128,
                 zslab128):
    f = functools.partial(
        pl.kernel,
        out_type=(jax.ShapeDtypeStruct((2, NP, 128), jnp.float32),
                  jax.ShapeDtypeStruct((2, NP, 128), jnp.float32)),
        mesh=_sc_mesh(),
        scratch_types=[
            pltpu.VMEM((NBH, B), jnp.int32),
            pltpu.VMEM((NBH, B), jnp.int32),
            pltpu.VMEM((B, 128), jnp.float32),
            pltpu.VMEM((B, 128), jnp.float32),
            pltpu.VMEM_SHARED((NP, 128), jnp.float32),
        ] + [pltpu.SemaphoreType.DMA] * 4,
    )(_scatter_kernel)
    return f(src_tiles2, src_tiles, dst_tiles, gflat, dinv128, zslab128)


def _mm_scale_kernel(x_ref, w1_ref, d0_ref, d1_ref, g_ref, dinv_ref):
    i = pl.program_id(0)
    h = jnp.dot(x_ref[...], w1_ref[...], preferred_element_type=jnp.float32)
    rows = lax.broadcasted_iota(jnp.int32, (BLK, 1), 0) + i * BLK
    deg = d0_ref[:, 0:1] + d1_ref[:, 0:1] + jnp.where(rows < N, 1.0, 0.0)
    dinv = jnp.where(deg > 0, lax.rsqrt(deg), 0.0)
    g = h * dinv
    g_ref[0] = g[:, :128]
    g_ref[1] = g[:, 128:]
    dinv_ref[...] = jnp.broadcast_to(dinv, (BLK, 128))


def _run_mm_scale(x_pad, w1, deg_parts):
    return pl.pallas_call(
        _mm_scale_kernel,
        grid=(NP // BLK,),
        in_specs=[
            pl.BlockSpec((BLK, D), lambda i: (i, 0)),
            pl.BlockSpec((D, H), lambda i: (0, 0)),
            pl.BlockSpec((BLK, 128), lambda i: (i, 0)),
            pl.BlockSpec((BLK, 128), lambda i: (i, 0)),
        ],
        out_specs=[
            pl.BlockSpec((2, BLK, 128), lambda i: (0, i, 0)),
            pl.BlockSpec((BLK, 128), lambda i: (i, 0)),
        ],
        out_shape=[jax.ShapeDtypeStruct((2, NP, 128), jnp.float32),
                   jax.ShapeDtypeStruct((NP, 128), jnp.float32)],
    )(x_pad, w1, deg_parts[0], deg_parts[1])


def _final_kernel(s0_ref, s1_ref, g0_ref, g1_ref, d0_ref, d1_ref,
                  u0_ref, u1_ref, b1_ref, w2_ref, b2_ref, y_ref, zacc):
    i = pl.program_id(0)

    @pl.when(i == 0)
    def _():
        zacc[...] = jnp.zeros_like(zacc)

    rows = lax.broadcasted_iota(jnp.int32, (BLK, 1), 0) + i * BLK
    valid = jnp.where(rows < N, 1.0, 0.0)
    deg = d0_ref[:, 0:1] + d1_ref[:, 0:1] + valid
    dinv = jnp.where(deg > 0, lax.rsqrt(deg), 0.0)
    u = u0_ref[:, 0:1] + u1_ref[:, 0:1]
    w = dinv * (u + dinv) * valid
    s_full = jnp.concatenate([s0_ref[...], s1_ref[...]], axis=1)
    g_full = jnp.concatenate([g0_ref[...], g1_ref[...]], axis=1)
    o = dinv * (s_full + g_full) + b1_ref[...]
    a = jnp.where(o > 0, o, jnp.exp(o) - 1.0)
    zacc[...] += jnp.sum(a * w, axis=0, keepdims=True)

    @pl.when(i == pl.num_programs(0) - 1)
    def _():
        y_ref[...] = (jnp.dot(zacc[...], w2_ref[...],
                              preferred_element_type=jnp.float32) * (1.0 / N)
                      + b2_ref[...])


def _run_final(s_halves, g0, g1, deg_parts, u_parts, b1r, w2, b2r):
    return pl.pallas_call(
        _final_kernel,
        grid=(NP // BLK,),
        in_specs=[
            pl.BlockSpec((BLK, 128), lambda i: (i, 0)),
            pl.BlockSpec((BLK, 128), lambda i: (i, 0)),
            pl.BlockSpec((BLK, 128), lambda i: (i, 0)),
            pl.BlockSpec((BLK, 128), lambda i: (i, 0)),
            pl.BlockSpec((BLK, 128), lambda i: (i, 0)),
            pl.BlockSpec((BLK, 128), lambda i: (i, 0)),
            pl.BlockSpec((BLK, 128), lambda i: (i, 0)),
            pl.BlockSpec((BLK, 128), lambda i: (i, 0)),
            pl.BlockSpec((1, H), lambda i: (0, 0)),
            pl.BlockSpec((H, H), lambda i: (0, 0)),
            pl.BlockSpec((1, H), lambda i: (0, 0)),
        ],
        out_specs=pl.BlockSpec((1, H), lambda i: (0, 0)),
        out_shape=jax.ShapeDtypeStruct((1, H), jnp.float32),
        scratch_shapes=[pltpu.VMEM((1, H), jnp.float32)],
        compiler_params=pltpu.CompilerParams(
            dimension_semantics=("arbitrary",)),
    )(s_halves[0], s_halves[1], g0, g1, deg_parts[0], deg_parts[1],
      u_parts[0], u_parts[1], b1r, w2, b2r)


def kernel(x, edge_index, W1, b1, W2, b2):
    f32 = jnp.float32
    x_pad = jnp.concatenate([x, jnp.zeros((NP - N, D), f32)], axis=0)
    src = edge_index[0]
    dst = edge_index[1]
    pad = jnp.full((EP - E,), TRASH, jnp.int32)
    src_tiles = jnp.concatenate([src, pad]).reshape(NTILES, NB, B)
    dst_tiles = jnp.concatenate([dst, pad]).reshape(NTILES, NB, B)

    ones128 = jnp.ones((B, 128), f32)
    zslab128 = jnp.zeros((SLAB, 128), f32)

    src_tiles2 = jnp.stack([src_tiles, src_tiles + NP])

    deg_parts = _run_deg(dst_tiles, ones128, zslab128)
    g_all, dinv128 = _run_mm_scale(x_pad, W1, deg_parts)
    gflat = g_all.reshape(2 * NP, 128)
    s_halves, u_parts = _run_scatter(src_tiles2, src_tiles, dst_tiles,
                                     gflat, dinv128, zslab128)
    y = _run_final(s_halves, g_all[0], g_all[1], deg_parts, u_parts,
                   b1.reshape(1, H), W2, b2.reshape(1, H))
    return y


# speedup vs baseline: 1.0015x; 1.0015x over previous
import functools

import jax
import jax.numpy as jnp
from jax import lax
from jax.experimental import pallas as pl
from jax.experimental.pallas import tpu as pltpu
from jax.experimental.pallas import tpu_sc as plsc

N = 10000
D = 256
H = 256
E = 160000

NP = 10240
EP = 163840
TRASH = NP - 1
NTILES = 16
TB = EP // NTILES
B = 128
NB = TB // B
NBH = NB // 2
SLAB = NP // NTILES
BLK = 512


def _sc_mesh():
    return plsc.VectorSubcoreMesh(core_axis_name="c", subcore_axis_name="s")


def _deg_kernel(dst_hbm, ones_hbm, zslab_hbm, deg_out, dst_v, ones_v, acc,
                ss0, ss1, ss2, ss3):
    c = lax.axis_index("c")
    s = lax.axis_index("s")
    pltpu.sync_copy(dst_hbm.at[s, pl.ds(c * NBH, NBH)], dst_v)
    pltpu.sync_copy(ones_hbm, ones_v)
    pltpu.sync_copy(zslab_hbm, acc.at[pl.ds(s * SLAB, SLAB)])
    plsc.subcore_barrier()

    pltpu.async_copy(ones_v, acc.at[dst_v.at[0]], ss0, add=True)
    pltpu.async_copy(ones_v, acc.at[dst_v.at[1]], ss1, add=True)
    pltpu.async_copy(ones_v, acc.at[dst_v.at[2]], ss2, add=True)
    pltpu.async_copy(ones_v, acc.at[dst_v.at[3]], ss3, add=True)

    def body(gi, carry):
        base = gi * 4
        pltpu.make_async_copy(ones_v, acc.at[dst_v.at[base]], ss0).wait()

        @pl.when(base + 4 < NBH)
        def _():
            pltpu.async_copy(ones_v, acc.at[dst_v.at[base + 4]], ss0,
                             add=True)

        pltpu.make_async_copy(ones_v, acc.at[dst_v.at[base + 1]], ss1).wait()

        @pl.when(base + 5 < NBH)
        def _():
            pltpu.async_copy(ones_v, acc.at[dst_v.at[base + 5]], ss1,
                             add=True)

        pltpu.make_async_copy(ones_v, acc.at[dst_v.at[base + 2]], ss2).wait()

        @pl.when(base + 6 < NBH)
        def _():
            pltpu.async_copy(ones_v, acc.at[dst_v.at[base + 6]], ss2,
                             add=True)

        pltpu.make_async_copy(ones_v, acc.at[dst_v.at[base + 3]], ss3).wait()

        @pl.when(base + 7 < NBH)
        def _():
            pltpu.async_copy(ones_v, acc.at[dst_v.at[base + 7]], ss3,
                             add=True)

        return carry

    lax.fori_loop(0, NBH // 4, body, 0)
    plsc.subcore_barrier()
    pltpu.sync_copy(acc.at[pl.ds(s * SLAB, SLAB)],
                    deg_out.at[c, pl.ds(s * SLAB, SLAB)])


def _run_deg(dst_tiles, ones128, zslab128):
    f = functools.partial(
        pl.kernel,
        out_type=jax.ShapeDtypeStruct((2, NP, 128), jnp.float32),
        mesh=_sc_mesh(),
        scratch_types=[
            pltpu.VMEM((NBH, B), jnp.int32),
            pltpu.VMEM((B, 128), jnp.float32),
            pltpu.VMEM_SHARED((NP, 128), jnp.float32),
            pltpu.SemaphoreType.DMA,
            pltpu.SemaphoreType.DMA,
            pltpu.SemaphoreType.DMA,
            pltpu.SemaphoreType.DMA,
        ],
    )(_deg_kernel)
    return f(dst_tiles, ones128, zslab128)


def _scatter_kernel(srco_hbm, src_hbm, dst_hbm, gflat_hbm, dinv_hbm,
                    zslab_hbm, s_out, u_out,
                    src_v, dst_v, r0, r1, acc, g0s, g1s, s0s, s1s):
    c = lax.axis_index("c")
    s = lax.axis_index("s")
    pltpu.sync_copy(zslab_hbm, acc.at[pl.ds(s * SLAB, SLAB)])
    plsc.subcore_barrier()

    for half in range(2):
        pltpu.sync_copy(srco_hbm.at[c, s, pl.ds(half * NBH, NBH)], src_v)
        pltpu.sync_copy(dst_hbm.at[s, pl.ds(half * NBH, NBH)], dst_v)

        pltpu.async_copy(gflat_hbm.at[src_v.at[0]], r0, g0s)
        pltpu.async_copy(gflat_hbm.at[src_v.at[1]], r1, g1s)

        def body(gi, carry):
            base = gi * 2
            pltpu.make_async_copy(gflat_hbm.at[src_v.at[base]], r0,
                                  g0s).wait()
            w0 = pltpu.async_copy(r0, acc.at[dst_v.at[base]], s0s, add=True)
            pltpu.make_async_copy(gflat_hbm.at[src_v.at[base + 1]], r1,
                                  g1s).wait()
            w1 = pltpu.async_copy(r1, acc.at[dst_v.at[base + 1]], s1s,
                                  add=True)
            w0.wait()

            @pl.when(base + 2 < NBH)
            def _():
                pltpu.async_copy(gflat_hbm.at[src_v.at[base + 2]], r0, g0s)

            w1.wait()

            @pl.when(base + 3 < NBH)
            def _():
                pltpu.async_copy(gflat_hbm.at[src_v.at[base + 3]], r1, g1s)

            return carry

        lax.fori_loop(0, NBH // 2, body, 0)

    plsc.subcore_barrier()
    pltpu.sync_copy(acc.at[pl.ds(s * SLAB, SLAB)],
                    s_out.at[c, pl.ds(s * SLAB, SLAB)])
    pltpu.sync_copy(zslab_hbm, acc.at[pl.ds(s * SLAB, SLAB)])
    plsc.subcore_barrier()

    pltpu.sync_copy(src_hbm.at[s, pl.ds(c * NBH, NBH)], src_v)
    pltpu.sync_copy(dst_hbm.at[s, pl.ds(c * NBH, NBH)], dst_v)

    pltpu.async_copy(dinv_hbm.at[dst_v.at[0]], r0, g0s)
    pltpu.async_copy(dinv_hbm.at[dst_v.at[1]], r1, g1s)

    def ubody(gi, carry):
        base = gi * 2
        pltpu.make_async_copy(dinv_hbm.at[dst_v.at[base]], r0, g0s).wait()
        w0 = pltpu.async_copy(r0, acc.at[src_v.at[base]], s0s, add=True)
        pltpu.make_async_copy(dinv_hbm.at[dst_v.at[base + 1]], r1,
                              g1s).wait()
        w1 = pltpu.async_copy(r1, acc.at[src_v.at[base + 1]], s1s,
                              add=True)
        w0.wait()

        @pl.when(base + 2 < NBH)
        def _():
            pltpu.async_copy(dinv_hbm.at[dst_v.at[base + 2]], r0, g0s)

        w1.wait()

        @pl.when(base + 3 < NBH)
        def _():
            pltpu.async_copy(dinv_hbm.at[dst_v.at[base + 3]], r1, g1s)

        return carry

    lax.fori_loop(0, NBH // 2, ubody, 0)
    plsc.subcore_barrier()
    pltpu.sync_copy(acc.at[pl.ds(s * SLAB, SLAB)],
                    u_out.at[c, pl.ds(s * SLAB, SLAB)])


def _run_scatter(src_tiles2, src_tiles, dst_tiles, gflat, dinv128,
                 zslab128):
    f = functools.partial(
        pl.kernel,
        out_type=(jax.ShapeDtypeStruct((2, NP, 128), jnp.float32),
                  jax.ShapeDtypeStruct((2, NP, 128), jnp.float32)),
        mesh=_sc_mesh(),
        scratch_types=[
            pltpu.VMEM((NBH, B), jnp.int32),
            pltpu.VMEM((NBH, B), jnp.int32),
            pltpu.VMEM((B, 128), jnp.float32),
            pltpu.VMEM((B, 128), jnp.float32),
            pltpu.VMEM_SHARED((NP, 128), jnp.float32),
        ] + [pltpu.SemaphoreType.DMA] * 4,
    )(_scatter_kernel)
    return f(src_tiles2, src_tiles, dst_tiles, gflat, dinv128, zslab128)


def _mm_scale_kernel(x_ref, w1_ref, d0_ref, d1_ref, g_ref, dinv_ref):
    i = pl.program_id(0)
    h = jnp.dot(x_ref[...], w1_ref[...], preferred_element_type=jnp.float32)
    rows = lax.broadcasted_iota(jnp.int32, (BLK, 1), 0) + i * BLK
    deg = d0_ref[:, 0:1] + d1_ref[:, 0:1] + jnp.where(rows < N, 1.0, 0.0)
    dinv = jnp.where(deg > 0, lax.rsqrt(deg), 0.0)
    g = h * dinv
    g_ref[0] = g[:, :128]
    g_ref[1] = g[:, 128:]
    dinv_ref[...] = jnp.broadcast_to(dinv, (BLK, 128))


def _run_mm_scale(x_pad, w1, deg_parts):
    return pl.pallas_call(
        _mm_scale_kernel,
        grid=(NP // BLK,),
        in_specs=[
            pl.BlockSpec((BLK, D), lambda i: (i, 0)),
            pl.BlockSpec((D, H), lambda i: (0, 0)),
            pl.BlockSpec((BLK, 128), lambda i: (i, 0)),
            pl.BlockSpec((BLK, 128), lambda i: (i, 0)),
        ],
        out_specs=[
            pl.BlockSpec((2, BLK, 128), lambda i: (0, i, 0)),
            pl.BlockSpec((BLK, 128), lambda i: (i, 0)),
        ],
        out_shape=[jax.ShapeDtypeStruct((2, NP, 128), jnp.float32),
                   jax.ShapeDtypeStruct((NP, 128), jnp.float32)],
    )(x_pad, w1, deg_parts[0], deg_parts[1])


def _final_kernel(s0_ref, s1_ref, g0_ref, g1_ref, d0_ref, d1_ref,
                  u0_ref, u1_ref, b1_ref, w2_ref, b2_ref, y_ref, zacc):
    i = pl.program_id(0)

    @pl.when(i == 0)
    def _():
        zacc[...] = jnp.zeros_like(zacc)

    rows = lax.broadcasted_iota(jnp.int32, (BLK, 1), 0) + i * BLK
    valid = jnp.where(rows < N, 1.0, 0.0)
    deg = d0_ref[:, 0:1] + d1_ref[:, 0:1] + valid
    dinv = jnp.where(deg > 0, lax.rsqrt(deg), 0.0)
    u = u0_ref[:, 0:1] + u1_ref[:, 0:1]
    w = dinv * (u + dinv) * valid
    s_full = jnp.concatenate([s0_ref[...], s1_ref[...]], axis=1)
    g_full = jnp.concatenate([g0_ref[...], g1_ref[...]], axis=1)
    o = dinv * (s_full + g_full) + b1_ref[...]
    a = jnp.where(o > 0, o, jnp.exp(o) - 1.0)
    zacc[...] += jnp.sum(a * w, axis=0, keepdims=True)

    @pl.when(i == pl.num_programs(0) - 1)
    def _():
        y_ref[...] = (jnp.dot(zacc[...], w2_ref[...],
                              preferred_element_type=jnp.float32) * (1.0 / N)
                      + b2_ref[...])


def _run_final(s_halves, g0, g1, deg_parts, u_parts, b1r, w2, b2r):
    return pl.pallas_call(
        _final_kernel,
        grid=(NP // BLK,),
        in_specs=[
            pl.BlockSpec((BLK, 128), lambda i: (i, 0)),
            pl.BlockSpec((BLK, 128), lambda i: (i, 0)),
            pl.BlockSpec((BLK, 128), lambda i: (i, 0)),
            pl.BlockSpec((BLK, 128), lambda i: (i, 0)),
            pl.BlockSpec((BLK, 128), lambda i: (i, 0)),
            pl.BlockSpec((BLK, 128), lambda i: (i, 0)),
            pl.BlockSpec((BLK, 128), lambda i: (i, 0)),
            pl.BlockSpec((BLK, 128), lambda i: (i, 0)),
            pl.BlockSpec((1, H), lambda i: (0, 0)),
            pl.BlockSpec((H, H), lambda i: (0, 0)),
            pl.BlockSpec((1, H), lambda i: (0, 0)),
        ],
        out_specs=pl.BlockSpec((1, H), lambda i: (0, 0)),
        out_shape=jax.ShapeDtypeStruct((1, H), jnp.float32),
        scratch_shapes=[pltpu.VMEM((1, H), jnp.float32)],
        compiler_params=pltpu.CompilerParams(
            dimension_semantics=("arbitrary",)),
    )(s_halves[0], s_halves[1], g0, g1, deg_parts[0], deg_parts[1],
      u_parts[0], u_parts[1], b1r, w2, b2r)


def kernel(x, edge_index, W1, b1, W2, b2):
    f32 = jnp.float32
    x_pad = jnp.concatenate([x, jnp.zeros((NP - N, D), f32)], axis=0)
    src = edge_index[0]
    dst = edge_index[1]
    pad = jnp.full((EP - E,), TRASH, jnp.int32)
    src_tiles = jnp.concatenate([src, pad]).reshape(NTILES, NB, B)
    dst_tiles = jnp.concatenate([dst, pad]).reshape(NTILES, NB, B)

    ones128 = jnp.ones((B, 128), f32)
    zslab128 = jnp.zeros((SLAB, 128), f32)

    src_tiles2 = jnp.stack([src_tiles, src_tiles + NP])

    deg_parts = _run_deg(dst_tiles, ones128, zslab128)
    g_all, dinv128 = _run_mm_scale(x_pad, W1, deg_parts)
    gflat = g_all.reshape(2 * NP, 128)
    s_halves, u_parts = _run_scatter(src_tiles2, src_tiles, dst_tiles,
                                     gflat, dinv128, zslab128)
    y = _run_final(s_halves, g_all[0], g_all[1], deg_parts, u_parts,
                   b1.reshape(1, H), W2, b2.reshape(1, H))
    return y


# speedup vs baseline: 1.0328x; 1.0312x over previous
import functools

import jax
import jax.numpy as jnp
from jax import lax
from jax.experimental import pallas as pl
from jax.experimental.pallas import tpu as pltpu
from jax.experimental.pallas import tpu_sc as plsc

N = 10000
D = 256
H = 256
E = 160000

NP = 10240
EP = 163840
TRASH = NP - 1
NTILES = 16
TB = EP // NTILES
B = 128
NB = TB // B
NBH = NB // 2
SLAB = NP // NTILES
BLK = 512


def _sc_mesh():
    return plsc.VectorSubcoreMesh(core_axis_name="c", subcore_axis_name="s")


def _deg_kernel(dst_hbm, ones_hbm, zslab_hbm, deg_out, dst_v, ones_v, acc,
                ss0, ss1, ss2, ss3):
    c = lax.axis_index("c")
    s = lax.axis_index("s")
    pltpu.sync_copy(dst_hbm.at[s, pl.ds(c * NBH, NBH)], dst_v)
    pltpu.sync_copy(ones_hbm, ones_v)
    pltpu.sync_copy(zslab_hbm, acc.at[pl.ds(s * SLAB, SLAB)])
    plsc.subcore_barrier()

    def body(j, carry):
        pltpu.sync_copy(ones_v, acc.at[dst_v.at[j]], add=True)
        return carry

    lax.fori_loop(0, NBH, body, 0)
    plsc.subcore_barrier()
    pltpu.sync_copy(acc.at[pl.ds(s * SLAB, SLAB)],
                    deg_out.at[c, pl.ds(s * SLAB, SLAB)])


def _run_deg(dst_tiles, ones128, zslab128):
    f = functools.partial(
        pl.kernel,
        out_type=jax.ShapeDtypeStruct((2, NP, 128), jnp.float32),
        mesh=_sc_mesh(),
        scratch_types=[
            pltpu.VMEM((NBH, B), jnp.int32),
            pltpu.VMEM((B, 128), jnp.float32),
            pltpu.VMEM_SHARED((NP, 128), jnp.float32),
            pltpu.SemaphoreType.DMA,
            pltpu.SemaphoreType.DMA,
            pltpu.SemaphoreType.DMA,
            pltpu.SemaphoreType.DMA,
        ],
    )(_deg_kernel)
    return f(dst_tiles, ones128, zslab128)


def _scatter_kernel(srco_hbm, src_hbm, dst_hbm, gflat_hbm, dinv_hbm,
                    zslab_hbm, s_out, u_out,
                    src_v, dst_v, r0, r1, acc, g0s, g1s, s0s, s1s):
    c = lax.axis_index("c")
    s = lax.axis_index("s")
    pltpu.sync_copy(zslab_hbm, acc.at[pl.ds(s * SLAB, SLAB)])
    plsc.subcore_barrier()

    for half in range(2):
        pltpu.sync_copy(srco_hbm.at[c, s, pl.ds(half * NBH, NBH)], src_v)
        pltpu.sync_copy(dst_hbm.at[s, pl.ds(half * NBH, NBH)], dst_v)

        pltpu.async_copy(gflat_hbm.at[src_v.at[0]], r0, g0s)
        pltpu.async_copy(gflat_hbm.at[src_v.at[1]], r1, g1s)

        def body(gi, carry):
            base = gi * 2
            pltpu.make_async_copy(gflat_hbm.at[src_v.at[base]], r0,
                                  g0s).wait()
            w0 = pltpu.async_copy(r0, acc.at[dst_v.at[base]], s0s, add=True)
            pltpu.make_async_copy(gflat_hbm.at[src_v.at[base + 1]], r1,
                                  g1s).wait()
            w0.wait()
            w1 = pltpu.async_copy(r1, acc.at[dst_v.at[base + 1]], s1s,
                                  add=True)

            @pl.when(base + 2 < NBH)
            def _():
                pltpu.async_copy(gflat_hbm.at[src_v.at[base + 2]], r0, g0s)

            w1.wait()

            @pl.when(base + 3 < NBH)
            def _():
                pltpu.async_copy(gflat_hbm.at[src_v.at[base + 3]], r1, g1s)

            return carry

        lax.fori_loop(0, NBH // 2, body, 0)

    plsc.subcore_barrier()
    pltpu.sync_copy(acc.at[pl.ds(s * SLAB, SLAB)],
                    s_out.at[c, pl.ds(s * SLAB, SLAB)])
    pltpu.sync_copy(zslab_hbm, acc.at[pl.ds(s * SLAB, SLAB)])
    plsc.subcore_barrier()

    pltpu.sync_copy(src_hbm.at[s, pl.ds(c * NBH, NBH)], src_v)
    pltpu.sync_copy(dst_hbm.at[s, pl.ds(c * NBH, NBH)], dst_v)

    pltpu.async_copy(dinv_hbm.at[dst_v.at[0]], r0, g0s)
    pltpu.async_copy(dinv_hbm.at[dst_v.at[1]], r1, g1s)

    def ubody(gi, carry):
        base = gi * 2
        pltpu.make_async_copy(dinv_hbm.at[dst_v.at[base]], r0, g0s).wait()
        w0 = pltpu.async_copy(r0, acc.at[src_v.at[base]], s0s, add=True)
        pltpu.make_async_copy(dinv_hbm.at[dst_v.at[base + 1]], r1,
                              g1s).wait()
        w0.wait()
        w1 = pltpu.async_copy(r1, acc.at[src_v.at[base + 1]], s1s,
                              add=True)

        @pl.when(base + 2 < NBH)
        def _():
            pltpu.async_copy(dinv_hbm.at[dst_v.at[base + 2]], r0, g0s)

        w1.wait()

        @pl.when(base + 3 < NBH)
        def _():
            pltpu.async_copy(dinv_hbm.at[dst_v.at[base + 3]], r1, g1s)

        return carry

    lax.fori_loop(0, NBH // 2, ubody, 0)
    plsc.subcore_barrier()
    pltpu.sync_copy(acc.at[pl.ds(s * SLAB, SLAB)],
                    u_out.at[c, pl.ds(s * SLAB, SLAB)])


def _run_scatter(src_tiles2, src_tiles, dst_tiles, gflat, dinv128,
                 zslab128):
    f = functools.partial(
        pl.kernel,
        out_type=(jax.ShapeDtypeStruct((2, NP, 128), jnp.float32),
                  jax.ShapeDtypeStruct((2, NP, 128), jnp.float32)),
        mesh=_sc_mesh(),
        scratch_types=[
            pltpu.VMEM((NBH, B), jnp.int32),
            pltpu.VMEM((NBH, B), jnp.int32),
            pltpu.VMEM((B, 128), jnp.float32),
            pltpu.VMEM((B, 128), jnp.float32),
            pltpu.VMEM_SHARED((NP, 128), jnp.float32),
        ] + [pltpu.SemaphoreType.DMA] * 4,
    )(_scatter_kernel)
    return f(src_tiles2, src_tiles, dst_tiles, gflat, dinv128, zslab128)


def _mm_scale_kernel(x_ref, w1_ref, d0_ref, d1_ref, g_ref, dinv_ref):
    i = pl.program_id(0)
    h = jnp.dot(x_ref[...], w1_ref[...], preferred_element_type=jnp.float32)
    rows = lax.broadcasted_iota(jnp.int32, (BLK, 1), 0) + i * BLK
    deg = d0_ref[:, 0:1] + d1_ref[:, 0:1] + jnp.where(rows < N, 1.0, 0.0)
    dinv = jnp.where(deg > 0, lax.rsqrt(deg), 0.0)
    g = h * dinv
    g_ref[0] = g[:, :128]
    g_ref[1] = g[:, 128:]
    dinv_ref[...] = jnp.broadcast_to(dinv, (BLK, 128))


def _run_mm_scale(x_pad, w1, deg_parts):
    return pl.pallas_call(
        _mm_scale_kernel,
        grid=(NP // BLK,),
        in_specs=[
            pl.BlockSpec((BLK, D), lambda i: (i, 0)),
            pl.BlockSpec((D, H), lambda i: (0, 0)),
            pl.BlockSpec((BLK, 128), lambda i: (i, 0)),
            pl.BlockSpec((BLK, 128), lambda i: (i, 0)),
        ],
        out_specs=[
            pl.BlockSpec((2, BLK, 128), lambda i: (0, i, 0)),
            pl.BlockSpec((BLK, 128), lambda i: (i, 0)),
        ],
        out_shape=[jax.ShapeDtypeStruct((2, NP, 128), jnp.float32),
                   jax.ShapeDtypeStruct((NP, 128), jnp.float32)],
    )(x_pad, w1, deg_parts[0], deg_parts[1])


def _final_kernel(s0_ref, s1_ref, g0_ref, g1_ref, d0_ref, d1_ref,
                  u0_ref, u1_ref, b1_ref, w2_ref, b2_ref, y_ref, zacc):
    i = pl.program_id(0)

    @pl.when(i == 0)
    def _():
        zacc[...] = jnp.zeros_like(zacc)

    rows = lax.broadcasted_iota(jnp.int32, (BLK, 1), 0) + i * BLK
    valid = jnp.where(rows < N, 1.0, 0.0)
    deg = d0_ref[:, 0:1] + d1_ref[:, 0:1] + valid
    dinv = jnp.where(deg > 0, lax.rsqrt(deg), 0.0)
    u = u0_ref[:, 0:1] + u1_ref[:, 0:1]
    w = dinv * (u + dinv) * valid
    s_full = jnp.concatenate([s0_ref[...], s1_ref[...]], axis=1)
    g_full = jnp.concatenate([g0_ref[...], g1_ref[...]], axis=1)
    o = dinv * (s_full + g_full) + b1_ref[...]
    a = jnp.where(o > 0, o, jnp.exp(o) - 1.0)
    zacc[...] += jnp.sum(a * w, axis=0, keepdims=True)

    @pl.when(i == pl.num_programs(0) - 1)
    def _():
        y_ref[...] = (jnp.dot(zacc[...], w2_ref[...],
                              preferred_element_type=jnp.float32) * (1.0 / N)
                      + b2_ref[...])


def _run_final(s_halves, g0, g1, deg_parts, u_parts, b1r, w2, b2r):
    return pl.pallas_call(
        _final_kernel,
        grid=(NP // BLK,),
        in_specs=[
            pl.BlockSpec((BLK, 128), lambda i: (i, 0)),
            pl.BlockSpec((BLK, 128), lambda i: (i, 0)),
            pl.BlockSpec((BLK, 128), lambda i: (i, 0)),
            pl.BlockSpec((BLK, 128), lambda i: (i, 0)),
            pl.BlockSpec((BLK, 128), lambda i: (i, 0)),
            pl.BlockSpec((BLK, 128), lambda i: (i, 0)),
            pl.BlockSpec((BLK, 128), lambda i: (i, 0)),
            pl.BlockSpec((BLK, 128), lambda i: (i, 0)),
            pl.BlockSpec((1, H), lambda i: (0, 0)),
            pl.BlockSpec((H, H), lambda i: (0, 0)),
            pl.BlockSpec((1, H), lambda i: (0, 0)),
        ],
        out_specs=pl.BlockSpec((1, H), lambda i: (0, 0)),
        out_shape=jax.ShapeDtypeStruct((1, H), jnp.float32),
        scratch_shapes=[pltpu.VMEM((1, H), jnp.float32)],
        compiler_params=pltpu.CompilerParams(
            dimension_semantics=("arbitrary",)),
    )(s_halves[0], s_halves[1], g0, g1, deg_parts[0], deg_parts[1],
      u_parts[0], u_parts[1], b1r, w2, b2r)


def kernel(x, edge_index, W1, b1, W2, b2):
    f32 = jnp.float32
    x_pad = jnp.concatenate([x, jnp.zeros((NP - N, D), f32)], axis=0)
    src = edge_index[0]
    dst = edge_index[1]
    pad = jnp.full((EP - E,), TRASH, jnp.int32)
    src_tiles = jnp.concatenate([src, pad]).reshape(NTILES, NB, B)
    dst_tiles = jnp.concatenate([dst, pad]).reshape(NTILES, NB, B)

    ones128 = jnp.ones((B, 128), f32)
    zslab128 = jnp.zeros((SLAB, 128), f32)

    src_tiles2 = jnp.stack([src_tiles, src_tiles + NP])

    deg_parts = _run_deg(dst_tiles, ones128, zslab128)
    g_all, dinv128 = _run_mm_scale(x_pad, W1, deg_parts)
    gflat = g_all.reshape(2 * NP, 128)
    s_halves, u_parts = _run_scatter(src_tiles2, src_tiles, dst_tiles,
                                     gflat, dinv128, zslab128)
    y = _run_final(s_halves, g_all[0], g_all[1], deg_parts, u_parts,
                   b1.reshape(1, H), W2, b2.reshape(1, H))
    return y
